# baseline (speedup 1.0000x reference)
import functools

import jax
import jax.numpy as jnp
from jax import lax
from jax.experimental import pallas as pl
from jax.experimental.pallas import tpu as pltpu
from jax.experimental.pallas import tpu_sc as plsc

N_NEIGHBORS = 32
EPSILON = 1e-5
DENOM_CONST = 1e-5
BIG = 3.0e38
CUT = 1.0e37

BLK = 2048
LANES = 128
GSUB = BLK // LANES
NSEL = 48
SC_LANES = 16


def _dist_kernel(q_ref, m_ref, d2_ref, gm_ref, sum_ref, tau_ref, acc_ref,
                 qmin_ref, *, n_q, n_k, n_blocks):
    j = pl.program_id(0)

    @pl.when(j == 0)
    def _init():
        acc_ref[0] = 0.0
        qmin_ref[...] = jnp.full((n_q, 256), BIG, jnp.float32)

    q = q_ref[...]
    m = m_ref[...]
    qm = lax.dot_general(q, m, (((1,), (1,)), ((), ())),
                         preferred_element_type=jnp.float32)
    q2 = jnp.sum(q * q, axis=1, keepdims=True)
    ones8 = jnp.ones((8, q.shape[1]), jnp.float32)
    m2row = lax.dot_general(ones8, m * m, (((1,), (1,)), ((), ())),
                            preferred_element_type=jnp.float32)[0:1, :]
    d2 = jnp.maximum(q2 + m2row - 2.0 * qm, 0.0)

    col = j * BLK + lax.broadcasted_iota(jnp.int32, (n_q, BLK), 1)
    valid = col < n_k
    acc_ref[0] += jnp.sum(jnp.where(valid, d2, 0.0))

    scd = jnp.where(valid, d2, BIG)
    d2_ref[...] = scd
    gm = scd[:, 0:LANES]
    for k in range(1, GSUB):
        gm = jnp.minimum(gm, scd[:, k * LANES:(k + 1) * LANES])
    gm_ref[...] = gm

    colpos = lax.broadcasted_iota(jnp.int32, (n_q, 256), 1)
    qs = qmin_ref[...]
    for k in range(4):
        qmk = jnp.min(scd[:, k * (BLK // 4):(k + 1) * (BLK // 4)], axis=1,
                      keepdims=True)
        qs = jnp.minimum(qs, jnp.where(colpos == j * 4 + k, qmk, BIG))
    qmin_ref[...] = qs

    @pl.when(j == n_blocks - 1)
    def _finish():
        sum_ref[...] = jnp.full((1, 1), acc_ref[0], jnp.float32)

        def tx(i, carry):
            qs, _ = carry
            mm = jnp.min(qs, axis=1, keepdims=True)
            return jnp.where(qs == mm, BIG, qs), mm

        _, tau = lax.fori_loop(0, N_NEIGHBORS, tx, (qmin_ref[...],
                                                    jnp.zeros((n_q, 1))))
        tau_ref[...] = tau


def _shuf_min(a, iota16):
    for s in (8, 4, 2, 1):
        a = jnp.minimum(a, jnp.take(a, iota16 ^ s))
    return a


def _shuf_max(a, iota16):
    for s in (8, 4, 2, 1):
        a = jnp.maximum(a, jnp.take(a, iota16 ^ s))
    return a


def _select_kernel(gm_hbm, tau_hbm, d2_hbm, out_hbm, gmv2, tausc, vtmp,
                   maskb, gbuf2, semg, sem1, semo, *, n_q, k_pad,
                   n_groups, rows_per_w):
    ngv = n_groups // SC_LANES
    wid = lax.axis_index("s") * 2 + lax.axis_index("c")
    big = jnp.full((SC_LANES,), BIG, jnp.float32)
    iota16 = lax.broadcasted_iota(jnp.int32, (SC_LANES,), 0)
    r0 = wid * rows_per_w

    pltpu.sync_copy(tau_hbm.at[pl.ds(r0, rows_per_w)], tausc)
    pltpu.make_async_copy(gm_hbm.at[r0], gmv2.at[0], semg).start()

    def row_body(rr, _):
        r = r0 + rr
        buf = rr & 1
        gmv = gmv2.at[buf]
        gbuf = gbuf2.at[buf]
        pltpu.make_async_copy(gm_hbm.at[r], gmv, semg).wait()

        @pl.when(rr + 1 < rows_per_w)
        def _prefetch():
            pltpu.make_async_copy(gm_hbm.at[r + 1], gmv2.at[1 - buf],
                                  semg).start()

        tvec = tausc[pl.ds((rr >> 4) * SC_LANES, SC_LANES)]
        rot = jnp.take(tvec, (iota16 + (rr & (SC_LANES - 1))) &
                       (SC_LANES - 1))
        tau_s = rot[0]

        @pl.when(rr >= 2)
        def _drain_prev_out():
            pltpu.make_async_copy(gbuf, out_hbm.at[r - 2], semo).wait()

        def initm(i, c):
            maskb[i, :] = big
            return c

        lax.fori_loop(0, NSEL, initm, 0)

        def handle_vreg(ci, base):
            for _ in range(2):
                vv = vtmp[...]
                mn = _shuf_min(vv, iota16)
                hit = (mn[0] <= tau_s) & (ci < NSEL)
                ci_c = jnp.minimum(ci, jnp.int32(NSEL - 1))

                @pl.when(hit)
                def _fire():
                    lanei = _shuf_min(jnp.where(vv == mn, iota16, SC_LANES),
                                      iota16)
                    gid = base + lanei[0]
                    jt = gid >> 7
                    lt = gid & (LANES - 1)
                    idxv = r * k_pad + jt * BLK + lt + LANES * iota16
                    maskb[ci_c, :] = mn
                    pltpu.make_async_copy(d2_hbm.at[idxv], gbuf.at[ci_c],
                                          sem1).start()
                    vtmp[...] = jnp.where(iota16 == lanei, big, vv)

                ci = ci + jnp.where(hit, jnp.int32(1), jnp.int32(0))
            return ci

        def p2(t, ci):
            v = [gmv[pl.ds((t * 4 + h) * SC_LANES, SC_LANES)]
                 for h in range(4)]
            mn = _shuf_min(jnp.minimum(jnp.minimum(v[0], v[1]),
                                       jnp.minimum(v[2], v[3])), iota16)

            def quad(ci):
                for h in range(4):
                    vtmp[...] = v[h]
                    ci = handle_vreg(ci, (t * 4 + h) * SC_LANES)
                return ci

            return lax.cond(mn[0] <= tau_s, quad, lambda c: c, ci)

        ci = lax.fori_loop(0, ngv // 4, p2, jnp.int32(0))
        nfired = jnp.minimum(ci, jnp.int32(NSEL))

        def drain(i, _):
            @pl.when(i < nfired)
            def _w():
                pltpu.make_async_copy(d2_hbm.at[pl.ds(0, SC_LANES)],
                                      gbuf.at[i], sem1).wait()
            return 0

        lax.fori_loop(0, NSEL, drain, 0)

        def fin(t, _):
            g = gbuf[t, :]
            mk = maskb[t, :]
            gbuf[t, :] = jnp.where(mk < CUT, g, big)
            return 0

        lax.fori_loop(0, NSEL, fin, 0)
        pltpu.make_async_copy(gbuf, out_hbm.at[r], semo).start()
        return 0

    lax.fori_loop(0, rows_per_w, row_body, 0)

    pltpu.make_async_copy(gbuf2.at[(rows_per_w - 2) & 1],
                          out_hbm.at[r0 + rows_per_w - 2], semo).wait()
    pltpu.make_async_copy(gbuf2.at[(rows_per_w - 1) & 1],
                          out_hbm.at[r0 + rows_per_w - 1], semo).wait()


def _final_kernel(cand_ref, sum_ref, out_ref, sc_ref, *, n_q, n_k):
    sc_ref[...] = cand_ref[...]
    mean = sum_ref[0, 0] / jnp.float32(n_q * n_k)
    topv = jnp.full((n_q, N_NEIGHBORS), BIG, jnp.float32)

    def body(i, topv):
        sc = sc_ref[...]
        mm = jnp.min(sc, axis=1, keepdims=True)
        shifted = jnp.concatenate(
            [jnp.full((n_q, 1), -BIG, jnp.float32), topv[:, :N_NEIGHBORS - 1]],
            axis=1)
        topv = jnp.minimum(topv, jnp.maximum(shifted, mm))
        sc_ref[...] = jnp.where(sc == mm, BIG, sc)
        return topv

    topv = lax.fori_loop(0, N_NEIGHBORS, body, topv)
    kv = EPSILON / (topv / mean + EPSILON)
    out_ref[...] = lax.rsqrt(jnp.sum(kv, axis=1, keepdims=True) + DENOM_CONST)


@jax.jit
def _episodic_reward(queries, memory):
    n_q, d = queries.shape
    n_k = memory.shape[0]
    n_blocks = pl.cdiv(n_k, BLK)
    k_pad = n_blocks * BLK
    n_groups = n_blocks * LANES
    mem_p = jnp.pad(memory, ((0, k_pad - n_k), (0, 0)))

    d2, gm, tot, tau = pl.pallas_call(
        functools.partial(_dist_kernel, n_q=n_q, n_k=n_k, n_blocks=n_blocks),
        grid=(n_blocks,),
        in_specs=[
            pl.BlockSpec((n_q, d), lambda j: (0, 0)),
            pl.BlockSpec((BLK, d), lambda j: (j, 0)),
        ],
        out_specs=[
            pl.BlockSpec((n_q, BLK), lambda j: (0, j)),
            pl.BlockSpec((n_q, LANES), lambda j: (0, j)),
            pl.BlockSpec((1, 1), lambda j: (0, 0)),
            pl.BlockSpec((n_q, 1), lambda j: (0, 0)),
        ],
        out_shape=[
            jax.ShapeDtypeStruct((n_q, k_pad), jnp.float32),
            jax.ShapeDtypeStruct((n_q, n_groups), jnp.float32),
            jax.ShapeDtypeStruct((1, 1), jnp.float32),
            jax.ShapeDtypeStruct((n_q, 1), jnp.float32),
        ],
        scratch_shapes=[pltpu.SMEM((1,), jnp.float32),
                        pltpu.VMEM((n_q, 256), jnp.float32)],
    )(queries, mem_p)

    rows_per_w = n_q // 32
    mesh = plsc.VectorSubcoreMesh(core_axis_name="c", subcore_axis_name="s")
    cand = pl.kernel(
        functools.partial(_select_kernel, n_q=n_q, k_pad=k_pad,
                          n_groups=n_groups, rows_per_w=rows_per_w),
        mesh=mesh,
        out_type=jax.ShapeDtypeStruct((n_q, NSEL, SC_LANES), jnp.float32),
        scratch_types=[
            pltpu.VMEM((2, n_groups), jnp.float32),
            pltpu.VMEM((rows_per_w,), jnp.float32),
            pltpu.VMEM((SC_LANES,), jnp.float32),
            pltpu.VMEM((NSEL, SC_LANES), jnp.float32),
            pltpu.VMEM((2, NSEL, SC_LANES), jnp.float32),
            pltpu.SemaphoreType.DMA,
            pltpu.SemaphoreType.DMA,
            pltpu.SemaphoreType.DMA,
        ],
    )(gm, tau.reshape(-1), d2.reshape(-1))

    out = pl.pallas_call(
        functools.partial(_final_kernel, n_q=n_q, n_k=n_k),
        in_specs=[
            pl.BlockSpec((n_q, NSEL * SC_LANES), lambda: (0, 0)),
            pl.BlockSpec(memory_space=pltpu.SMEM),
        ],
        out_specs=pl.BlockSpec((n_q, 1), lambda: (0, 0)),
        out_shape=jax.ShapeDtypeStruct((n_q, 1), jnp.float32),
        scratch_shapes=[pltpu.VMEM((n_q, NSEL * SC_LANES), jnp.float32)],
    )(cand.reshape(n_q, NSEL * SC_LANES), tot)
    return out[:, 0]


def kernel(queries, memory):
    return _episodic_reward(queries, memory)


# speedup vs baseline: 1.0791x; 1.0791x over previous
import functools

import jax
import jax.numpy as jnp
from jax import lax
from jax.experimental import pallas as pl
from jax.experimental.pallas import tpu as pltpu
from jax.experimental.pallas import tpu_sc as plsc

N_NEIGHBORS = 32
EPSILON = 1e-5
DENOM_CONST = 1e-5
BIG = 3.0e38
CUT = 1.0e37

BLK = 2048
LANES = 128
GSUB = BLK // LANES
NSEL = 40
CAPV = 160
N_CHUNKS = 4
SC_LANES = 16


def _dist_kernel(q_ref, m_ref, d2_ref, gm_ref, sum_ref, tau_ref, acc_ref,
                 qmin_ref, *, n_q, n_k, n_blocks):
    j = pl.program_id(0)

    @pl.when(j == 0)
    def _init():
        acc_ref[0] = 0.0
        qmin_ref[...] = jnp.full((n_q, 256), BIG, jnp.float32)

    q = q_ref[...]
    m = m_ref[...]
    qm = lax.dot_general(q, m, (((1,), (1,)), ((), ())),
                         preferred_element_type=jnp.float32)
    q2 = jnp.sum(q * q, axis=1, keepdims=True)
    ones8 = jnp.ones((8, q.shape[1]), jnp.float32)
    m2row = lax.dot_general(ones8, m * m, (((1,), (1,)), ((), ())),
                            preferred_element_type=jnp.float32)[0:1, :]
    d2 = jnp.maximum(q2 + m2row - 2.0 * qm, 0.0)

    col = j * BLK + lax.broadcasted_iota(jnp.int32, (n_q, BLK), 1)
    valid = col < n_k
    acc_ref[0] += jnp.sum(jnp.where(valid, d2, 0.0))

    scd = jnp.where(valid, d2, BIG)
    d2_ref[...] = scd
    gm = scd[:, 0:LANES]
    for k in range(1, GSUB):
        gm = jnp.minimum(gm, scd[:, k * LANES:(k + 1) * LANES])
    gm_ref[...] = gm

    colpos = lax.broadcasted_iota(jnp.int32, (n_q, 256), 1)
    qs = qmin_ref[...]
    for k in range(4):
        qmk = jnp.min(scd[:, k * (BLK // 4):(k + 1) * (BLK // 4)], axis=1,
                      keepdims=True)
        qs = jnp.minimum(qs, jnp.where(colpos == j * 4 + k, qmk, BIG))
    qmin_ref[...] = qs

    @pl.when(j == n_blocks - 1)
    def _finish():
        sum_ref[...] = jnp.full((1, 1), acc_ref[0], jnp.float32)

        def tx(i, carry):
            qs, _ = carry
            mm = jnp.min(qs, axis=1, keepdims=True)
            return jnp.where(qs == mm, BIG, qs), mm

        _, tau = lax.fori_loop(0, N_NEIGHBORS, tx, (qmin_ref[...],
                                                    jnp.zeros((n_q, 1))))
        tau_ref[...] = tau


def _shuf_min(a, iota16):
    for s in (8, 4, 2, 1):
        a = jnp.minimum(a, jnp.take(a, iota16 ^ s))
    return a


def _shuf_max(a, iota16):
    for s in (8, 4, 2, 1):
        a = jnp.maximum(a, jnp.take(a, iota16 ^ s))
    return a


def _select_kernel(gm_hbm, tau_hbm, d2_hbm, out_hbm, gmv2, tausc, cvals,
                   cids, summ, maskb, gbuf2, semg, sem1, semo, *, n_q, k_pad,
                   n_groups, rows_per_w):
    ngv = n_groups // SC_LANES
    wid = lax.axis_index("s") * 2 + lax.axis_index("c")
    big = jnp.full((SC_LANES,), BIG, jnp.float32)
    iota16 = lax.broadcasted_iota(jnp.int32, (SC_LANES,), 0)
    r0 = wid * rows_per_w

    pltpu.sync_copy(tau_hbm.at[pl.ds(r0, rows_per_w)],
                    tausc.at[pl.ds(0, rows_per_w)])
    pltpu.make_async_copy(gm_hbm.at[r0], gmv2.at[0], semg).start()

    def row_body(rr, _):
        r = r0 + rr
        buf = rr & 1
        gmv = gmv2.at[buf]
        gbuf = gbuf2.at[buf]
        pltpu.make_async_copy(gm_hbm.at[r], gmv, semg).wait()

        @pl.when(rr + 1 < rows_per_w)
        def _prefetch():
            pltpu.make_async_copy(gm_hbm.at[r + 1], gmv2.at[1 - buf],
                                  semg).start()

        tvec = tausc[pl.ds((rr >> 4) * SC_LANES, SC_LANES)]
        rot = jnp.take(tvec, (iota16 + (rr & (SC_LANES - 1))) &
                       (SC_LANES - 1))
        tau_s = rot[0]

        @pl.when(rr >= 2)
        def _drain_prev_out():
            pltpu.make_async_copy(gbuf, out_hbm.at[r - 2], semo).wait()

        def inits(t, c):
            summ[pl.ds(t * SC_LANES, SC_LANES)] = big
            return c

        lax.fori_loop(0, CAPV // SC_LANES, inits, 0)

        def summ_set(e, valsplat):
            si = (e >> 4) * SC_LANES
            sl = e & (SC_LANES - 1)
            old = summ[pl.ds(si, SC_LANES)]
            summ[pl.ds(si, SC_LANES)] = jnp.where(iota16 == sl, valsplat,
                                                  old)

        def p2(t, ev):
            v = [gmv[pl.ds((t * 4 + h) * SC_LANES, SC_LANES)]
                 for h in range(4)]
            mn = _shuf_min(jnp.minimum(jnp.minimum(v[0], v[1]),
                                       jnp.minimum(v[2], v[3])), iota16)

            def app(ev):
                e = jnp.minimum(ev, jnp.int32(CAPV - 4))
                o = e * SC_LANES
                for h in range(4):
                    cvals[pl.ds(o + h * SC_LANES, SC_LANES)] = v[h]
                    cids[pl.ds(o + h * SC_LANES, SC_LANES)] = \
                        (t * 4 + h) * SC_LANES + iota16
                    summ_set(e + h, _shuf_min(v[h], iota16))
                return ev + 4

            return lax.cond(mn[0] <= tau_s, app, lambda e: e, ev)

        lax.fori_loop(0, ngv // 4, p2, jnp.int32(0))

        nsum = CAPV // SC_LANES

        def p3(i, _):
            def mintree(t, acc):
                return jnp.minimum(acc, summ[pl.ds(t * SC_LANES, SC_LANES)])

            mv = lax.fori_loop(0, nsum, mintree, big)
            msv = _shuf_min(mv, iota16)

            def findev(t, acc):
                sv = summ[pl.ds(t * SC_LANES, SC_LANES)]
                eids = t * SC_LANES + iota16
                return jnp.minimum(acc,
                                   jnp.where(sv == msv, eids,
                                             jnp.int32(9999)))

            evv = lax.fori_loop(0, nsum, findev,
                                jnp.full((SC_LANES,), 9999, jnp.int32))
            e = jnp.minimum(_shuf_min(evv, iota16)[0], CAPV - 1)
            o = e * SC_LANES
            bv = cvals[pl.ds(o, SC_LANES)]
            lanev = _shuf_min(jnp.where(bv == msv, iota16, SC_LANES), iota16)
            gid = jnp.take(cids[pl.ds(o, SC_LANES)], lanev)[0]
            gid = jnp.clip(gid, 0, n_groups - 1)
            bv2 = jnp.where(iota16 == lanev, big, bv)
            cvals[pl.ds(o, SC_LANES)] = bv2
            summ_set(e, _shuf_min(bv2, iota16))
            jt = gid >> 7
            lt = gid & (LANES - 1)
            idxv = r * k_pad + jt * BLK + lt + LANES * iota16
            maskb[i, :] = msv
            pltpu.make_async_copy(d2_hbm.at[idxv], gbuf.at[i], sem1).start()
            return 0

        lax.fori_loop(0, NSEL, p3, 0)

        def drain(i, _):
            pltpu.make_async_copy(d2_hbm.at[pl.ds(0, SC_LANES)],
                                  gbuf.at[i], sem1).wait()
            return 0

        lax.fori_loop(0, NSEL, drain, 0)

        def fin(t, _):
            g = gbuf[t, :]
            mk = maskb[t, :]
            gbuf[t, :] = jnp.where(mk < CUT, g, big)
            return 0

        lax.fori_loop(0, NSEL, fin, 0)
        pltpu.make_async_copy(gbuf, out_hbm.at[r], semo).start()
        return 0

    lax.fori_loop(0, rows_per_w, row_body, 0)

    pltpu.make_async_copy(gbuf2.at[(rows_per_w - 2) & 1],
                          out_hbm.at[r0 + rows_per_w - 2], semo).wait()
    pltpu.make_async_copy(gbuf2.at[(rows_per_w - 1) & 1],
                          out_hbm.at[r0 + rows_per_w - 1], semo).wait()


def _final_kernel(cand_ref, sum_ref, out_ref, sc_ref, *, n_q, n_k):
    sc_ref[...] = cand_ref[...]
    mean = sum_ref[0, 0] / jnp.float32(n_q * n_k)
    topv = jnp.full((n_q, N_NEIGHBORS), BIG, jnp.float32)

    def body(i, topv):
        sc = sc_ref[...]
        mm = jnp.min(sc, axis=1, keepdims=True)
        shifted = jnp.concatenate(
            [jnp.full((n_q, 1), -BIG, jnp.float32), topv[:, :N_NEIGHBORS - 1]],
            axis=1)
        topv = jnp.minimum(topv, jnp.maximum(shifted, mm))
        sc_ref[...] = jnp.where(sc == mm, BIG, sc)
        return topv

    topv = lax.fori_loop(0, N_NEIGHBORS, body, topv)
    kv = EPSILON / (topv / mean + EPSILON)
    out_ref[...] = lax.rsqrt(jnp.sum(kv, axis=1, keepdims=True) + DENOM_CONST)


@jax.jit
def _episodic_reward(queries, memory):
    n_q, d = queries.shape
    n_k = memory.shape[0]
    n_blocks = pl.cdiv(n_k, BLK)
    k_pad = n_blocks * BLK
    n_groups = n_blocks * LANES
    mem_p = jnp.pad(memory, ((0, k_pad - n_k), (0, 0)))

    nq_c = n_q // N_CHUNKS
    rows_per_w = nq_c // 32
    mesh = plsc.VectorSubcoreMesh(core_axis_name="c", subcore_axis_name="s")

    cands, tots = [], []
    for c in range(N_CHUNKS):
        qc = queries[c * nq_c:(c + 1) * nq_c]
        d2, gm, tot, tau = pl.pallas_call(
            functools.partial(_dist_kernel, n_q=nq_c, n_k=n_k,
                              n_blocks=n_blocks),
            grid=(n_blocks,),
            in_specs=[
                pl.BlockSpec((nq_c, d), lambda j: (0, 0)),
                pl.BlockSpec((BLK, d), lambda j: (j, 0)),
            ],
            out_specs=[
                pl.BlockSpec((nq_c, BLK), lambda j: (0, j)),
                pl.BlockSpec((nq_c, LANES), lambda j: (0, j)),
                pl.BlockSpec((1, 1), lambda j: (0, 0)),
                pl.BlockSpec((nq_c, 1), lambda j: (0, 0)),
            ],
            out_shape=[
                jax.ShapeDtypeStruct((nq_c, k_pad), jnp.float32),
                jax.ShapeDtypeStruct((nq_c, n_groups), jnp.float32),
                jax.ShapeDtypeStruct((1, 1), jnp.float32),
                jax.ShapeDtypeStruct((nq_c, 1), jnp.float32),
            ],
            scratch_shapes=[pltpu.SMEM((1,), jnp.float32),
                            pltpu.VMEM((nq_c, 256), jnp.float32)],
        )(qc, mem_p)

        cand = pl.kernel(
            functools.partial(_select_kernel, n_q=nq_c, k_pad=k_pad,
                              n_groups=n_groups, rows_per_w=rows_per_w),
            mesh=mesh,
            out_type=jax.ShapeDtypeStruct((nq_c, NSEL, SC_LANES),
                                          jnp.float32),
            scratch_types=[
                pltpu.VMEM((2, n_groups), jnp.float32),
                pltpu.VMEM((max(rows_per_w, SC_LANES),), jnp.float32),
                pltpu.VMEM((CAPV * SC_LANES,), jnp.float32),
                pltpu.VMEM((CAPV * SC_LANES,), jnp.int32),
                pltpu.VMEM((CAPV,), jnp.float32),
                pltpu.VMEM((NSEL, SC_LANES), jnp.float32),
                pltpu.VMEM((2, NSEL, SC_LANES), jnp.float32),
                pltpu.SemaphoreType.DMA,
                pltpu.SemaphoreType.DMA,
                pltpu.SemaphoreType.DMA,
            ],
        )(gm, tau.reshape(-1), d2.reshape(-1))
        cands.append(cand.reshape(nq_c, NSEL * SC_LANES))
        tots.append(tot)

    tot = tots[0]
    for t in tots[1:]:
        tot = tot + t
    cand_all = jnp.concatenate(cands, axis=0)

    out = pl.pallas_call(
        functools.partial(_final_kernel, n_q=n_q, n_k=n_k),
        in_specs=[
            pl.BlockSpec((n_q, NSEL * SC_LANES), lambda: (0, 0)),
            pl.BlockSpec(memory_space=pltpu.SMEM),
        ],
        out_specs=pl.BlockSpec((n_q, 1), lambda: (0, 0)),
        out_shape=jax.ShapeDtypeStruct((n_q, 1), jnp.float32),
        scratch_shapes=[pltpu.VMEM((n_q, NSEL * SC_LANES), jnp.float32)],
    )(cand_all, tot)
    return out[:, 0]


def kernel(queries, memory):
    return _episodic_reward(queries, memory)


# speedup vs baseline: 1.1160x; 1.0342x over previous
import functools

import jax
import jax.numpy as jnp
from jax import lax
from jax.experimental import pallas as pl
from jax.experimental.pallas import tpu as pltpu
from jax.experimental.pallas import tpu_sc as plsc

N_NEIGHBORS = 32
EPSILON = 1e-5
DENOM_CONST = 1e-5
BIG = 3.0e38
CUT = 1.0e37

BLK = 2048
LANES = 128
GSUB = BLK // LANES
NSEL = 40
CAPV = 160
N_CHUNKS = 1
SC_LANES = 16


def _dist_kernel(q_ref, m_ref, d2_ref, gm_ref, sum_ref, tau_ref, acc_ref,
                 qmin_ref, *, n_q, n_k, n_blocks):
    j = pl.program_id(0)

    @pl.when(j == 0)
    def _init():
        acc_ref[0] = 0.0
        qmin_ref[...] = jnp.full((n_q, 256), BIG, jnp.float32)

    q = q_ref[...]
    m = m_ref[...]
    qm = lax.dot_general(q.astype(jnp.bfloat16), m.astype(jnp.bfloat16),
                         (((1,), (1,)), ((), ())),
                         preferred_element_type=jnp.float32)
    q2 = jnp.sum(q * q, axis=1, keepdims=True)
    ones8 = jnp.ones((8, q.shape[1]), jnp.float32)
    m2row = lax.dot_general(ones8, m * m, (((1,), (1,)), ((), ())),
                            preferred_element_type=jnp.float32)[0:1, :]
    d2 = jnp.maximum(q2 + m2row - 2.0 * qm, 0.0)

    col = j * BLK + lax.broadcasted_iota(jnp.int32, (n_q, BLK), 1)
    valid = col < n_k
    acc_ref[0] += jnp.sum(jnp.where(valid, d2, 0.0))

    scd = jnp.where(valid, d2, BIG)
    d2_ref[...] = scd
    gm = scd[:, 0:LANES]
    for k in range(1, GSUB):
        gm = jnp.minimum(gm, scd[:, k * LANES:(k + 1) * LANES])
    gm_ref[...] = gm

    colpos = lax.broadcasted_iota(jnp.int32, (n_q, 256), 1)
    qs = qmin_ref[...]
    for k in range(4):
        qmk = jnp.min(scd[:, k * (BLK // 4):(k + 1) * (BLK // 4)], axis=1,
                      keepdims=True)
        qs = jnp.minimum(qs, jnp.where(colpos == j * 4 + k, qmk, BIG))
    qmin_ref[...] = qs

    @pl.when(j == n_blocks - 1)
    def _finish():
        sum_ref[...] = jnp.full((1, 1), acc_ref[0], jnp.float32)

        def tx(i, carry):
            qs, _ = carry
            mm = jnp.min(qs, axis=1, keepdims=True)
            return jnp.where(qs == mm, BIG, qs), mm

        _, tau = lax.fori_loop(0, N_NEIGHBORS, tx, (qmin_ref[...],
                                                    jnp.zeros((n_q, 1))))
        tau_ref[...] = tau


def _shuf_min(a, iota16):
    for s in (8, 4, 2, 1):
        a = jnp.minimum(a, jnp.take(a, iota16 ^ s))
    return a


def _shuf_max(a, iota16):
    for s in (8, 4, 2, 1):
        a = jnp.maximum(a, jnp.take(a, iota16 ^ s))
    return a


def _select_kernel(gm_hbm, tau_hbm, d2_hbm, out_hbm, gmv2, tausc, cvals,
                   cids, summ, maskb, gbuf2, semg, sem1, semo, *, n_q, k_pad,
                   n_groups, rows_per_w):
    ngv = n_groups // SC_LANES
    wid = lax.axis_index("s") * 2 + lax.axis_index("c")
    big = jnp.full((SC_LANES,), BIG, jnp.float32)
    iota16 = lax.broadcasted_iota(jnp.int32, (SC_LANES,), 0)
    r0 = wid * rows_per_w

    pltpu.sync_copy(tau_hbm.at[pl.ds(r0, rows_per_w)],
                    tausc.at[pl.ds(0, rows_per_w)])
    pltpu.make_async_copy(gm_hbm.at[r0], gmv2.at[0], semg).start()

    def row_body(rr, _):
        r = r0 + rr
        buf = rr & 1
        gmv = gmv2.at[buf]
        gbuf = gbuf2.at[buf]
        pltpu.make_async_copy(gm_hbm.at[r], gmv, semg).wait()

        @pl.when(rr + 1 < rows_per_w)
        def _prefetch():
            pltpu.make_async_copy(gm_hbm.at[r + 1], gmv2.at[1 - buf],
                                  semg).start()

        tvec = tausc[pl.ds((rr >> 4) * SC_LANES, SC_LANES)]
        rot = jnp.take(tvec, (iota16 + (rr & (SC_LANES - 1))) &
                       (SC_LANES - 1))
        tau_s = rot[0]

        @pl.when(rr >= 2)
        def _drain_prev_out():
            pltpu.make_async_copy(gbuf, out_hbm.at[r - 2], semo).wait()

        def inits(t, c):
            summ[pl.ds(t * SC_LANES, SC_LANES)] = big
            return c

        lax.fori_loop(0, CAPV // SC_LANES, inits, 0)

        def summ_set(e, valsplat):
            si = (e >> 4) * SC_LANES
            sl = e & (SC_LANES - 1)
            old = summ[pl.ds(si, SC_LANES)]
            summ[pl.ds(si, SC_LANES)] = jnp.where(iota16 == sl, valsplat,
                                                  old)

        def p2(t, ev):
            v = [gmv[pl.ds((t * 4 + h) * SC_LANES, SC_LANES)]
                 for h in range(4)]
            mn = _shuf_min(jnp.minimum(jnp.minimum(v[0], v[1]),
                                       jnp.minimum(v[2], v[3])), iota16)

            def app(ev):
                e = jnp.minimum(ev, jnp.int32(CAPV - 4))
                o = e * SC_LANES
                for h in range(4):
                    cvals[pl.ds(o + h * SC_LANES, SC_LANES)] = v[h]
                    cids[pl.ds(o + h * SC_LANES, SC_LANES)] = \
                        (t * 4 + h) * SC_LANES + iota16
                    summ_set(e + h, _shuf_min(v[h], iota16))
                return ev + 4

            return lax.cond(mn[0] <= tau_s, app, lambda e: e, ev)

        lax.fori_loop(0, ngv // 4, p2, jnp.int32(0))

        nsum = CAPV // SC_LANES

        def p3(i, _):
            def mintree(t, acc):
                return jnp.minimum(acc, summ[pl.ds(t * SC_LANES, SC_LANES)])

            mv = lax.fori_loop(0, nsum, mintree, big)
            msv = _shuf_min(mv, iota16)

            def findev(t, acc):
                sv = summ[pl.ds(t * SC_LANES, SC_LANES)]
                eids = t * SC_LANES + iota16
                return jnp.minimum(acc,
                                   jnp.where(sv == msv, eids,
                                             jnp.int32(9999)))

            evv = lax.fori_loop(0, nsum, findev,
                                jnp.full((SC_LANES,), 9999, jnp.int32))
            e = jnp.minimum(_shuf_min(evv, iota16)[0], CAPV - 1)
            o = e * SC_LANES
            bv = cvals[pl.ds(o, SC_LANES)]
            lanev = _shuf_min(jnp.where(bv == msv, iota16, SC_LANES), iota16)
            gid = jnp.take(cids[pl.ds(o, SC_LANES)], lanev)[0]
            gid = jnp.clip(gid, 0, n_groups - 1)
            bv2 = jnp.where(iota16 == lanev, big, bv)
            cvals[pl.ds(o, SC_LANES)] = bv2
            summ_set(e, _shuf_min(bv2, iota16))
            jt = gid >> 7
            lt = gid & (LANES - 1)
            idxv = r * k_pad + jt * BLK + lt + LANES * iota16
            maskb[i, :] = msv
            pltpu.make_async_copy(d2_hbm.at[idxv], gbuf.at[i], sem1).start()
            return 0

        lax.fori_loop(0, NSEL, p3, 0)

        def drain(i, _):
            pltpu.make_async_copy(d2_hbm.at[pl.ds(0, SC_LANES)],
                                  gbuf.at[i], sem1).wait()
            return 0

        lax.fori_loop(0, NSEL, drain, 0)

        def fin(t, _):
            g = gbuf[t, :]
            mk = maskb[t, :]
            gbuf[t, :] = jnp.where(mk < CUT, g, big)
            return 0

        lax.fori_loop(0, NSEL, fin, 0)
        pltpu.make_async_copy(gbuf, out_hbm.at[r], semo).start()
        return 0

    lax.fori_loop(0, rows_per_w, row_body, 0)

    pltpu.make_async_copy(gbuf2.at[(rows_per_w - 2) & 1],
                          out_hbm.at[r0 + rows_per_w - 2], semo).wait()
    pltpu.make_async_copy(gbuf2.at[(rows_per_w - 1) & 1],
                          out_hbm.at[r0 + rows_per_w - 1], semo).wait()


def _final_kernel(cand_ref, sum_ref, out_ref, sc_ref, *, n_q, n_k):
    sc_ref[...] = cand_ref[...]
    mean = sum_ref[0, 0] / jnp.float32(n_q * n_k)
    topv = jnp.full((n_q, N_NEIGHBORS), BIG, jnp.float32)

    def body(i, topv):
        sc = sc_ref[...]
        mm = jnp.min(sc, axis=1, keepdims=True)
        shifted = jnp.concatenate(
            [jnp.full((n_q, 1), -BIG, jnp.float32), topv[:, :N_NEIGHBORS - 1]],
            axis=1)
        topv = jnp.minimum(topv, jnp.maximum(shifted, mm))
        sc_ref[...] = jnp.where(sc == mm, BIG, sc)
        return topv

    topv = lax.fori_loop(0, N_NEIGHBORS, body, topv)
    kv = EPSILON / (topv / mean + EPSILON)
    out_ref[...] = lax.rsqrt(jnp.sum(kv, axis=1, keepdims=True) + DENOM_CONST)


@jax.jit
def _episodic_reward(queries, memory):
    n_q, d = queries.shape
    n_k = memory.shape[0]
    n_blocks = pl.cdiv(n_k, BLK)
    k_pad = n_blocks * BLK
    n_groups = n_blocks * LANES
    mem_p = jnp.pad(memory, ((0, k_pad - n_k), (0, 0)))

    nq_c = n_q // N_CHUNKS
    rows_per_w = nq_c // 32
    mesh = plsc.VectorSubcoreMesh(core_axis_name="c", subcore_axis_name="s")

    cands, tots = [], []
    for c in range(N_CHUNKS):
        qc = queries[c * nq_c:(c + 1) * nq_c]
        d2, gm, tot, tau = pl.pallas_call(
            functools.partial(_dist_kernel, n_q=nq_c, n_k=n_k,
                              n_blocks=n_blocks),
            grid=(n_blocks,),
            in_specs=[
                pl.BlockSpec((nq_c, d), lambda j: (0, 0)),
                pl.BlockSpec((BLK, d), lambda j: (j, 0)),
            ],
            out_specs=[
                pl.BlockSpec((nq_c, BLK), lambda j: (0, j)),
                pl.BlockSpec((nq_c, LANES), lambda j: (0, j)),
                pl.BlockSpec((1, 1), lambda j: (0, 0)),
                pl.BlockSpec((nq_c, 1), lambda j: (0, 0)),
            ],
            out_shape=[
                jax.ShapeDtypeStruct((nq_c, k_pad), jnp.float32),
                jax.ShapeDtypeStruct((nq_c, n_groups), jnp.float32),
                jax.ShapeDtypeStruct((1, 1), jnp.float32),
                jax.ShapeDtypeStruct((nq_c, 1), jnp.float32),
            ],
            scratch_shapes=[pltpu.SMEM((1,), jnp.float32),
                            pltpu.VMEM((nq_c, 256), jnp.float32)],
        )(qc, mem_p)

        cand = pl.kernel(
            functools.partial(_select_kernel, n_q=nq_c, k_pad=k_pad,
                              n_groups=n_groups, rows_per_w=rows_per_w),
            mesh=mesh,
            out_type=jax.ShapeDtypeStruct((nq_c, NSEL, SC_LANES),
                                          jnp.float32),
            scratch_types=[
                pltpu.VMEM((2, n_groups), jnp.float32),
                pltpu.VMEM((max(rows_per_w, SC_LANES),), jnp.float32),
                pltpu.VMEM((CAPV * SC_LANES,), jnp.float32),
                pltpu.VMEM((CAPV * SC_LANES,), jnp.int32),
                pltpu.VMEM((CAPV,), jnp.float32),
                pltpu.VMEM((NSEL, SC_LANES), jnp.float32),
                pltpu.VMEM((2, NSEL, SC_LANES), jnp.float32),
                pltpu.SemaphoreType.DMA,
                pltpu.SemaphoreType.DMA,
                pltpu.SemaphoreType.DMA,
            ],
        )(gm, tau.reshape(-1), d2.reshape(-1))
        cands.append(cand.reshape(nq_c, NSEL * SC_LANES))
        tots.append(tot)

    tot = tots[0]
    for t in tots[1:]:
        tot = tot + t
    cand_all = jnp.concatenate(cands, axis=0)

    out = pl.pallas_call(
        functools.partial(_final_kernel, n_q=n_q, n_k=n_k),
        in_specs=[
            pl.BlockSpec((n_q, NSEL * SC_LANES), lambda: (0, 0)),
            pl.BlockSpec(memory_space=pltpu.SMEM),
        ],
        out_specs=pl.BlockSpec((n_q, 1), lambda: (0, 0)),
        out_shape=jax.ShapeDtypeStruct((n_q, 1), jnp.float32),
        scratch_shapes=[pltpu.VMEM((n_q, NSEL * SC_LANES), jnp.float32)],
    )(cand_all, tot)
    return out[:, 0]


def kernel(queries, memory):
    return _episodic_reward(queries, memory)


# speedup vs baseline: 1.1349x; 1.0169x over previous
import functools

import jax
import jax.numpy as jnp
from jax import lax
from jax.experimental import pallas as pl
from jax.experimental.pallas import tpu as pltpu
from jax.experimental.pallas import tpu_sc as plsc

N_NEIGHBORS = 32
EPSILON = 1e-5
DENOM_CONST = 1e-5
BIG = 3.0e38
CUT = 1.0e37

BLK = 2048
LANES = 128
GSUB = BLK // LANES
NSEL = 36
CAPV = 160
N_CHUNKS = 1
SC_LANES = 16


def _dist_kernel(q_ref, m_ref, d2_ref, gm_ref, sum_ref, tau_ref, acc_ref,
                 qmin_ref, *, n_q, n_k, n_blocks):
    j = pl.program_id(0)

    @pl.when(j == 0)
    def _init():
        acc_ref[0] = 0.0
        qmin_ref[...] = jnp.full((n_q, 256), BIG, jnp.float32)

    q = q_ref[...]
    m = m_ref[...]
    qm = lax.dot_general(q.astype(jnp.bfloat16), m.astype(jnp.bfloat16),
                         (((1,), (1,)), ((), ())),
                         preferred_element_type=jnp.float32)
    q2 = jnp.sum(q * q, axis=1, keepdims=True)
    ones8 = jnp.ones((8, q.shape[1]), jnp.float32)
    m2row = lax.dot_general(ones8, m * m, (((1,), (1,)), ((), ())),
                            preferred_element_type=jnp.float32)[0:1, :]
    d2 = jnp.maximum(q2 + m2row - 2.0 * qm, 0.0)

    col = j * BLK + lax.broadcasted_iota(jnp.int32, (n_q, BLK), 1)
    valid = col < n_k
    acc_ref[0] += jnp.sum(jnp.where(valid, d2, 0.0))

    scd = jnp.where(valid, d2, BIG)
    d2_ref[...] = scd
    gm = scd[:, 0:LANES]
    for k in range(1, GSUB):
        gm = jnp.minimum(gm, scd[:, k * LANES:(k + 1) * LANES])
    gm_ref[...] = gm

    colpos = lax.broadcasted_iota(jnp.int32, (n_q, 256), 1)
    qs = qmin_ref[...]
    for k in range(4):
        qmk = jnp.min(scd[:, k * (BLK // 4):(k + 1) * (BLK // 4)], axis=1,
                      keepdims=True)
        qs = jnp.minimum(qs, jnp.where(colpos == j * 4 + k, qmk, BIG))
    qmin_ref[...] = qs

    @pl.when(j == n_blocks - 1)
    def _finish():
        sum_ref[...] = jnp.full((1, 1), acc_ref[0], jnp.float32)

        def tx(i, carry):
            qs, _ = carry
            mm = jnp.min(qs, axis=1, keepdims=True)
            return jnp.where(qs == mm, BIG, qs), mm

        _, tau = lax.fori_loop(0, N_NEIGHBORS, tx, (qmin_ref[...],
                                                    jnp.zeros((n_q, 1))))
        tau_ref[...] = tau


def _shuf_min(a, iota16):
    for s in (8, 4, 2, 1):
        a = jnp.minimum(a, jnp.take(a, iota16 ^ s))
    return a


def _shuf_max(a, iota16):
    for s in (8, 4, 2, 1):
        a = jnp.maximum(a, jnp.take(a, iota16 ^ s))
    return a


def _select_kernel(gm_hbm, tau_hbm, d2_hbm, out_hbm, gmv2, tausc, cvals,
                   cids, summ, maskb, gbuf2, semg, sem1, semo, *, n_q, k_pad,
                   n_groups, rows_per_w):
    ngv = n_groups // SC_LANES
    wid = lax.axis_index("s") * 2 + lax.axis_index("c")
    big = jnp.full((SC_LANES,), BIG, jnp.float32)
    iota16 = lax.broadcasted_iota(jnp.int32, (SC_LANES,), 0)
    r0 = wid * rows_per_w

    pltpu.sync_copy(tau_hbm.at[pl.ds(r0, rows_per_w)],
                    tausc.at[pl.ds(0, rows_per_w)])
    pltpu.make_async_copy(gm_hbm.at[r0], gmv2.at[0], semg).start()

    def row_body(rr, _):
        r = r0 + rr
        buf = rr & 1
        gmv = gmv2.at[buf]
        gbuf = gbuf2.at[buf]
        pltpu.make_async_copy(gm_hbm.at[r], gmv, semg).wait()

        @pl.when(rr + 1 < rows_per_w)
        def _prefetch():
            pltpu.make_async_copy(gm_hbm.at[r + 1], gmv2.at[1 - buf],
                                  semg).start()

        tvec = tausc[pl.ds((rr >> 4) * SC_LANES, SC_LANES)]
        rot = jnp.take(tvec, (iota16 + (rr & (SC_LANES - 1))) &
                       (SC_LANES - 1))
        tau_s = rot[0]

        @pl.when(rr >= 2)
        def _drain_prev_out():
            pltpu.make_async_copy(gbuf, out_hbm.at[r - 2], semo).wait()

        def inits(t, c):
            summ[pl.ds(t * SC_LANES, SC_LANES)] = big
            return c

        lax.fori_loop(0, CAPV // SC_LANES, inits, 0)

        def summ_set(e, valsplat):
            si = (e >> 4) * SC_LANES
            sl = e & (SC_LANES - 1)
            old = summ[pl.ds(si, SC_LANES)]
            summ[pl.ds(si, SC_LANES)] = jnp.where(iota16 == sl, valsplat,
                                                  old)

        def p2(t, ev):
            v = [gmv[pl.ds((t * 4 + h) * SC_LANES, SC_LANES)]
                 for h in range(4)]
            mn = _shuf_min(jnp.minimum(jnp.minimum(v[0], v[1]),
                                       jnp.minimum(v[2], v[3])), iota16)

            def app(ev):
                e = jnp.minimum(ev, jnp.int32(CAPV - 4))
                o = e * SC_LANES
                for h in range(4):
                    cvals[pl.ds(o + h * SC_LANES, SC_LANES)] = v[h]
                    cids[pl.ds(o + h * SC_LANES, SC_LANES)] = \
                        (t * 4 + h) * SC_LANES + iota16
                    summ_set(e + h, _shuf_min(v[h], iota16))
                return ev + 4

            return lax.cond(mn[0] <= tau_s, app, lambda e: e, ev)

        lax.fori_loop(0, ngv // 4, p2, jnp.int32(0))

        nsum = CAPV // SC_LANES

        def p3(i, _):
            def mintree(t, acc):
                return jnp.minimum(acc, summ[pl.ds(t * SC_LANES, SC_LANES)])

            mv = lax.fori_loop(0, nsum, mintree, big)
            msv = _shuf_min(mv, iota16)

            def findev(t, acc):
                sv = summ[pl.ds(t * SC_LANES, SC_LANES)]
                eids = t * SC_LANES + iota16
                return jnp.minimum(acc,
                                   jnp.where(sv == msv, eids,
                                             jnp.int32(9999)))

            evv = lax.fori_loop(0, nsum, findev,
                                jnp.full((SC_LANES,), 9999, jnp.int32))
            e = jnp.minimum(_shuf_min(evv, iota16)[0], CAPV - 1)
            o = e * SC_LANES
            bv = cvals[pl.ds(o, SC_LANES)]
            lanev = _shuf_min(jnp.where(bv == msv, iota16, SC_LANES), iota16)
            gid = jnp.take(cids[pl.ds(o, SC_LANES)], lanev)[0]
            gid = jnp.clip(gid, 0, n_groups - 1)
            bv2 = jnp.where(iota16 == lanev, big, bv)
            cvals[pl.ds(o, SC_LANES)] = bv2
            summ_set(e, _shuf_min(bv2, iota16))
            jt = gid >> 7
            lt = gid & (LANES - 1)
            idxv = r * k_pad + jt * BLK + lt + LANES * iota16
            maskb[i, :] = msv
            pltpu.make_async_copy(d2_hbm.at[idxv], gbuf.at[i], sem1).start()
            return 0

        lax.fori_loop(0, NSEL, p3, 0)

        def drain(i, _):
            pltpu.make_async_copy(d2_hbm.at[pl.ds(0, SC_LANES)],
                                  gbuf.at[i], sem1).wait()
            return 0

        lax.fori_loop(0, NSEL, drain, 0)

        def fin(t, _):
            g = gbuf[t, :]
            mk = maskb[t, :]
            gbuf[t, :] = jnp.where(mk < CUT, g, big)
            return 0

        lax.fori_loop(0, NSEL, fin, 0)
        pltpu.make_async_copy(gbuf, out_hbm.at[r], semo).start()
        return 0

    lax.fori_loop(0, rows_per_w, row_body, 0)

    pltpu.make_async_copy(gbuf2.at[(rows_per_w - 2) & 1],
                          out_hbm.at[r0 + rows_per_w - 2], semo).wait()
    pltpu.make_async_copy(gbuf2.at[(rows_per_w - 1) & 1],
                          out_hbm.at[r0 + rows_per_w - 1], semo).wait()


def _final_kernel(cand_ref, sum_ref, out_ref, sc_ref, *, n_q, n_k):
    sc_ref[...] = cand_ref[...]
    mean = sum_ref[0, 0] / jnp.float32(n_q * n_k)
    topv = jnp.full((n_q, N_NEIGHBORS), BIG, jnp.float32)

    def body(i, topv):
        sc = sc_ref[...]
        mm = jnp.min(sc, axis=1, keepdims=True)
        shifted = jnp.concatenate(
            [jnp.full((n_q, 1), -BIG, jnp.float32), topv[:, :N_NEIGHBORS - 1]],
            axis=1)
        topv = jnp.minimum(topv, jnp.maximum(shifted, mm))
        sc_ref[...] = jnp.where(sc == mm, BIG, sc)
        return topv

    topv = lax.fori_loop(0, N_NEIGHBORS, body, topv)
    kv = EPSILON / (topv / mean + EPSILON)
    out_ref[...] = lax.rsqrt(jnp.sum(kv, axis=1, keepdims=True) + DENOM_CONST)


@jax.jit
def _episodic_reward(queries, memory):
    n_q, d = queries.shape
    n_k = memory.shape[0]
    n_blocks = pl.cdiv(n_k, BLK)
    k_pad = n_blocks * BLK
    n_groups = n_blocks * LANES
    mem_p = jnp.pad(memory, ((0, k_pad - n_k), (0, 0)))

    nq_c = n_q // N_CHUNKS
    rows_per_w = nq_c // 32
    mesh = plsc.VectorSubcoreMesh(core_axis_name="c", subcore_axis_name="s")

    cands, tots = [], []
    for c in range(N_CHUNKS):
        qc = queries[c * nq_c:(c + 1) * nq_c]
        d2, gm, tot, tau = pl.pallas_call(
            functools.partial(_dist_kernel, n_q=nq_c, n_k=n_k,
                              n_blocks=n_blocks),
            grid=(n_blocks,),
            in_specs=[
                pl.BlockSpec((nq_c, d), lambda j: (0, 0)),
                pl.BlockSpec((BLK, d), lambda j: (j, 0)),
            ],
            out_specs=[
                pl.BlockSpec((nq_c, BLK), lambda j: (0, j)),
                pl.BlockSpec((nq_c, LANES), lambda j: (0, j)),
                pl.BlockSpec((1, 1), lambda j: (0, 0)),
                pl.BlockSpec((nq_c, 1), lambda j: (0, 0)),
            ],
            out_shape=[
                jax.ShapeDtypeStruct((nq_c, k_pad), jnp.float32),
                jax.ShapeDtypeStruct((nq_c, n_groups), jnp.float32),
                jax.ShapeDtypeStruct((1, 1), jnp.float32),
                jax.ShapeDtypeStruct((nq_c, 1), jnp.float32),
            ],
            scratch_shapes=[pltpu.SMEM((1,), jnp.float32),
                            pltpu.VMEM((nq_c, 256), jnp.float32)],
        )(qc, mem_p)

        cand = pl.kernel(
            functools.partial(_select_kernel, n_q=nq_c, k_pad=k_pad,
                              n_groups=n_groups, rows_per_w=rows_per_w),
            mesh=mesh,
            out_type=jax.ShapeDtypeStruct((nq_c, NSEL, SC_LANES),
                                          jnp.float32),
            scratch_types=[
                pltpu.VMEM((2, n_groups), jnp.float32),
                pltpu.VMEM((max(rows_per_w, SC_LANES),), jnp.float32),
                pltpu.VMEM((CAPV * SC_LANES,), jnp.float32),
                pltpu.VMEM((CAPV * SC_LANES,), jnp.int32),
                pltpu.VMEM((CAPV,), jnp.float32),
                pltpu.VMEM((NSEL, SC_LANES), jnp.float32),
                pltpu.VMEM((2, NSEL, SC_LANES), jnp.float32),
                pltpu.SemaphoreType.DMA,
                pltpu.SemaphoreType.DMA,
                pltpu.SemaphoreType.DMA,
            ],
        )(gm, tau.reshape(-1), d2.reshape(-1))
        cands.append(cand.reshape(nq_c, NSEL * SC_LANES))
        tots.append(tot)

    tot = tots[0]
    for t in tots[1:]:
        tot = tot + t
    cand_all = jnp.concatenate(cands, axis=0)

    out = pl.pallas_call(
        functools.partial(_final_kernel, n_q=n_q, n_k=n_k),
        in_specs=[
            pl.BlockSpec((n_q, NSEL * SC_LANES), lambda: (0, 0)),
            pl.BlockSpec(memory_space=pltpu.SMEM),
        ],
        out_specs=pl.BlockSpec((n_q, 1), lambda: (0, 0)),
        out_shape=jax.ShapeDtypeStruct((n_q, 1), jnp.float32),
        scratch_shapes=[pltpu.VMEM((n_q, NSEL * SC_LANES), jnp.float32)],
    )(cand_all, tot)
    return out[:, 0]


def kernel(queries, memory):
    return _episodic_reward(queries, memory)
